# baseline (speedup 1.0000x reference)
import functools

import jax
import jax.numpy as jnp
from jax import lax
from jax.experimental import pallas as pl
from jax.experimental.pallas import tpu as pltpu
from jax.experimental.pallas import tpu_sc as plsc

_CHUNK = 32


def _make_sc_kernel(batch, seq_len, dim, num_workers, chunk):
    rows_per_worker = seq_len // num_workers
    n = rows_per_worker // chunk

    def body(w_hbm, out_hbm, buf, sem):
        wid = lax.axis_index("s") * 2 + lax.axis_index("c")
        base = wid * rows_per_worker
        for i in range(n):
            rows = pl.ds(base + i * chunk, chunk)
            pltpu.sync_copy(w_hbm.at[rows, :], buf)
            copies = [
                pltpu.make_async_copy(buf, out_hbm.at[b, rows, :], sem)
                for b in range(batch)
            ]
            for c in copies:
                c.start()
            for c in copies:
                c.wait()

    return body


def kernel(input_ids, emb_weight):
    batch, seq_len = input_ids.shape
    dim = emb_weight.shape[1]
    info = plsc.get_sparse_core_info()
    num_workers = info.num_cores * info.num_subcores
    chunk = _CHUNK
    mesh = plsc.VectorSubcoreMesh(core_axis_name="c", subcore_axis_name="s")
    k = functools.partial(
        pl.kernel,
        mesh=mesh,
        out_type=jax.ShapeDtypeStruct((batch, seq_len, dim), emb_weight.dtype),
        scratch_types=[
            pltpu.VMEM((chunk, dim), jnp.float32),
            pltpu.SemaphoreType.DMA,
        ],
    )(_make_sc_kernel(batch, seq_len, dim, num_workers, chunk))
    return k(emb_weight)


# speedup vs baseline: 1.0543x; 1.0543x over previous
import functools

import jax
import jax.numpy as jnp
from jax import lax
from jax.experimental import pallas as pl
from jax.experimental.pallas import tpu as pltpu
from jax.experimental.pallas import tpu_sc as plsc

_CHUNK = 32


def _make_sc_kernel(batch, seq_len, dim, num_workers, chunk):
    rows_per_worker = seq_len // num_workers
    n = rows_per_worker // chunk

    def body(w_hbm, out_hbm, buf, sem):
        wid = lax.axis_index("s") * 2 + lax.axis_index("c")
        base = wid * rows_per_worker

        def rows(i):
            return pl.ds(base + i * chunk, chunk)

        pltpu.sync_copy(w_hbm.at[rows(0), :], buf.at[0])
        for i in range(n):
            copies = [
                pltpu.make_async_copy(
                    buf.at[i % 2], out_hbm.at[b, rows(i), :], sem
                )
                for b in range(batch)
            ]
            for c in copies:
                c.start()
            if i + 1 < n:
                pltpu.sync_copy(w_hbm.at[rows(i + 1), :], buf.at[(i + 1) % 2])
            for c in copies:
                c.wait()

    return body


def kernel(input_ids, emb_weight):
    batch, seq_len = input_ids.shape
    dim = emb_weight.shape[1]
    info = plsc.get_sparse_core_info()
    num_workers = info.num_cores * info.num_subcores
    chunk = _CHUNK
    mesh = plsc.VectorSubcoreMesh(core_axis_name="c", subcore_axis_name="s")
    k = functools.partial(
        pl.kernel,
        mesh=mesh,
        out_type=jax.ShapeDtypeStruct((batch, seq_len, dim), emb_weight.dtype),
        scratch_types=[
            pltpu.VMEM((2, chunk, dim), jnp.float32),
            pltpu.SemaphoreType.DMA,
        ],
    )(_make_sc_kernel(batch, seq_len, dim, num_workers, chunk))
    return k(emb_weight)
